# baseline (speedup 1.0000x reference)
import functools

import jax
import jax.numpy as jnp
from jax import lax
from jax.experimental import pallas as pl
from jax.experimental.pallas import tpu as pltpu
from jax.experimental.pallas import tpu_sc as plsc

EMB = 256
LANES = 16
PAIR = 160
SCH = (96, 80)
TCH = (PAIR - SCH[0], PAIR - SCH[1])
SMAX = max(SCH)
TMAX = max(TCH)
NUM_TILES = 32


def _prep_body(w0_ref, w1_ref, w2_ref, a0_ref, a1_ref, a2_ref, t_ref, c_ref):
    for a0 in range(w0_ref.shape[0]):
        for a1 in range(w1_ref.shape[0]):
            for a2 in range(w2_ref.shape[0]):
                c = (a0 * w1_ref.shape[0] + a1) * w2_ref.shape[0] + a2
                t_ref[0, c, :] = w0_ref[a0, :] + w1_ref[a1, :] + w2_ref[a2, :]
    for k in range(1, NUM_TILES):
        t_ref[k, :, :] = t_ref[0, :, :]
    n1 = w1_ref.shape[0]
    n2 = w2_ref.shape[0]
    c_ref[...] = (a0_ref[...] * n1 + a1_ref[...]) * n2 + a2_ref[...]


def _make_sc_kernel(num_edges, ncombo):
    npairs = num_edges // PAIR
    base_cnt = npairs // NUM_TILES
    rem = npairs % NUM_TILES
    iters = base_cnt + (1 if rem else 0)
    strip = iters * PAIR
    mesh = plsc.VectorSubcoreMesh(core_axis_name="c", subcore_axis_name="s")

    @functools.partial(
        pl.kernel,
        mesh=mesh,
        out_type=jax.ShapeDtypeStruct((num_edges, EMB), jnp.float32),
        scratch_types=[
            pltpu.VMEM((strip,), jnp.int32),
            pltpu.VMEM((2 * SMAX, EMB), jnp.float32),
            pltpu.VMEM((2 * TMAX, EMB), jnp.float32),
            pltpu.VMEM((ncombo * EMB,), jnp.float32),
            pltpu.SemaphoreType.DMA,
            pltpu.SemaphoreType.DMA,
            pltpu.SemaphoreType.DMA,
            pltpu.SemaphoreType.DMA,
            pltpu.SemaphoreType.DMA,
            pltpu.SemaphoreType.DMA,
        ],
    )
    def sc_build(t_rep_hbm, t_flat_hbm, c_hbm, out_hbm, idx_v, sbuf, tbuf,
                 t_v, g_sem0, g_sem1, ws_sem0, ws_sem1, wt_sem0, wt_sem1):
        cid = lax.axis_index("c")
        sid = lax.axis_index("s")
        w = sid * 2 + cid

        start = w * base_cnt + jnp.minimum(w, rem)
        count = jnp.where(w < rem, base_cnt + 1, base_cnt)

        pltpu.sync_copy(t_flat_hbm, t_v)

        if rem:
            @pl.when(w < rem)
            def _():
                pltpu.sync_copy(
                    c_hbm.at[pl.ds(start * PAIR, (base_cnt + 1) * PAIR)],
                    idx_v)

            @pl.when(w >= rem)
            def _():
                pltpu.sync_copy(
                    c_hbm.at[pl.ds(start * PAIR, base_cnt * PAIR)],
                    idx_v.at[pl.ds(0, base_cnt * PAIR)])
        else:
            pltpu.sync_copy(c_hbm.at[pl.ds(start * PAIR, base_cnt * PAIR)],
                            idx_v)

        off = w * ncombo

        @plsc.parallel_loop(0, strip // LANES, unroll=4)
        def rebase(k):
            idx_v[pl.ds(k * LANES, LANES)] = (
                idx_v[pl.ds(k * LANES, LANES)] + off)

        g_sems = (g_sem0, g_sem1)
        ws_sems = (ws_sem0, ws_sem1)
        wt_sems = (wt_sem0, wt_sem1)

        def gather_start(p, par):
            pltpu.async_copy(
                t_rep_hbm.at[idx_v.at[pl.ds(p * PAIR, SCH[par])]],
                sbuf.at[pl.ds(par * SMAX, SCH[par]), :], g_sems[par])

        def gather_wait(par):
            pltpu.make_async_copy(
                t_rep_hbm.at[idx_v.at[pl.ds(0, SCH[par])]],
                sbuf.at[pl.ds(par * SMAX, SCH[par]), :], g_sems[par]).wait()

        def swrite_start(p, par):
            pltpu.async_copy(
                sbuf.at[pl.ds(par * SMAX, SCH[par]), :],
                out_hbm.at[pl.ds((start + p) * PAIR, SCH[par]), :],
                ws_sems[par])

        def swrite_wait(par):
            pltpu.make_async_copy(sbuf.at[pl.ds(0, SCH[par]), :],
                                  out_hbm.at[pl.ds(0, SCH[par]), :],
                                  ws_sems[par]).wait()

        def build_chunk(p, par):
            @plsc.parallel_loop(0, TCH[par] // LANES, unroll=1)
            def grp(g):
                cv = (idx_v[pl.ds(p * PAIR + SCH[par] + g * LANES, LANES)]
                      - off) * EMB
                rbase = par * TMAX + g * LANES
                for k in range(LANES):
                    c = cv[k]
                    vals = [t_v[pl.ds(c + j * LANES, LANES)]
                            for j in range(EMB // LANES)]
                    for j, v in enumerate(vals):
                        tbuf[rbase + k, pl.ds(j * LANES, LANES)] = v

        def twrite_start(p, par):
            pltpu.async_copy(
                tbuf.at[pl.ds(par * TMAX, TCH[par]), :],
                out_hbm.at[pl.ds((start + p) * PAIR + SCH[par], TCH[par]), :],
                wt_sems[par])

        def twrite_wait(par):
            pltpu.make_async_copy(tbuf.at[pl.ds(0, TCH[par]), :],
                                  out_hbm.at[pl.ds(0, TCH[par]), :],
                                  wt_sems[par]).wait()

        gather_start(0, 0)

        def body(j, carry):
            even = lax.rem(j, 2) == 0

            def lane_work(par):
                @pl.when(j >= 1)
                def _():
                    swrite_wait(1 - par)

                @pl.when(j + 1 < count)
                def _():
                    gather_start(j + 1, 1 - par)

                @pl.when(j >= 2)
                def _():
                    twrite_wait(par)

                build_chunk(j, par)
                twrite_start(j, par)
                gather_wait(par)
                swrite_start(j, par)

            @pl.when(jnp.logical_and(j < count, even))
            def _():
                lane_work(0)

            @pl.when(jnp.logical_and(j < count, jnp.logical_not(even)))
            def _():
                lane_work(1)

            return carry

        lax.fori_loop(0, iters, body, 0)

        last_par = lax.rem(count - 1, 2)

        @pl.when(last_par == 0)
        def _():
            swrite_wait(0)

        @pl.when(last_par == 1)
        def _():
            swrite_wait(1)

        twrite_wait(0)
        twrite_wait(1)

    return sc_build


def kernel(edge_attr, W0, W1, W2):
    num_edges = edge_attr.shape[0]
    attr = edge_attr.astype(jnp.int32)
    rows = num_edges // PAIR
    a0 = attr[:, 0].reshape(rows, PAIR)
    a1 = attr[:, 1].reshape(rows, PAIR)
    a2 = attr[:, 2].reshape(rows, PAIR)

    ncombo = W0.shape[0] * W1.shape[0] * W2.shape[0]
    t_rep, c2d = pl.pallas_call(
        _prep_body,
        out_shape=(
            jax.ShapeDtypeStruct((NUM_TILES, ncombo, EMB), jnp.float32),
            jax.ShapeDtypeStruct((rows, PAIR), jnp.int32),
        ),
    )(W0, W1, W2, a0, a1, a2)

    return _make_sc_kernel(num_edges, ncombo)(
        t_rep.reshape(NUM_TILES * ncombo, EMB),
        t_rep[0].reshape(ncombo * EMB),
        c2d.reshape(num_edges))


# speedup vs baseline: 1.0523x; 1.0523x over previous
import functools

import jax
import jax.numpy as jnp
from jax import lax
from jax.experimental import pallas as pl
from jax.experimental.pallas import tpu as pltpu
from jax.experimental.pallas import tpu_sc as plsc

EMB = 256
LANES = 16
CHUNK = 80
NUM_TILES = 32


def _prep_body(w0_ref, w1_ref, w2_ref, a0_ref, a1_ref, a2_ref, t_ref, c_ref):
    for a0 in range(w0_ref.shape[0]):
        for a1 in range(w1_ref.shape[0]):
            for a2 in range(w2_ref.shape[0]):
                c = (a0 * w1_ref.shape[0] + a1) * w2_ref.shape[0] + a2
                t_ref[0, c, :] = w0_ref[a0, :] + w1_ref[a1, :] + w2_ref[a2, :]
    for k in range(1, NUM_TILES):
        t_ref[k, :, :] = t_ref[0, :, :]
    n1 = w1_ref.shape[0]
    n2 = w2_ref.shape[0]
    c_ref[...] = (a0_ref[...] * n1 + a1_ref[...]) * n2 + a2_ref[...]


def _make_sc_kernel(num_edges, ncombo):
    nchunks = num_edges // CHUNK
    base_cnt = nchunks // NUM_TILES
    rem = nchunks % NUM_TILES
    iters = base_cnt + (1 if rem else 0)
    iters2 = (iters + 1) // 2
    strip = iters * CHUNK
    mesh = plsc.VectorSubcoreMesh(core_axis_name="c", subcore_axis_name="s")

    @functools.partial(
        pl.kernel,
        mesh=mesh,
        out_type=jax.ShapeDtypeStruct((num_edges, EMB), jnp.float32),
        scratch_types=[
            pltpu.VMEM((strip,), jnp.int32),
            pltpu.VMEM((2 * CHUNK, EMB), jnp.float32),
            pltpu.VMEM((2 * CHUNK, EMB), jnp.float32),
            pltpu.VMEM((ncombo * EMB,), jnp.float32),
            pltpu.SemaphoreType.DMA,
            pltpu.SemaphoreType.DMA,
            pltpu.SemaphoreType.DMA,
        ],
    )
    def sc_build(t_rep_hbm, t_flat_hbm, c_hbm, out_hbm, idx_v, sbuf, tbuf,
                 t_v, g_sem, ws_sem, wt_sem):
        cid = lax.axis_index("c")
        sid = lax.axis_index("s")
        w = sid * 2 + cid

        start = w * base_cnt + jnp.minimum(w, rem)
        count = jnp.where(w < rem, base_cnt + 1, base_cnt)

        pltpu.sync_copy(t_flat_hbm, t_v)

        if rem:
            @pl.when(w < rem)
            def _():
                pltpu.sync_copy(
                    c_hbm.at[pl.ds(start * CHUNK, (base_cnt + 1) * CHUNK)],
                    idx_v)

            @pl.when(w >= rem)
            def _():
                pltpu.sync_copy(
                    c_hbm.at[pl.ds(start * CHUNK, base_cnt * CHUNK)],
                    idx_v.at[pl.ds(0, base_cnt * CHUNK)])
        else:
            pltpu.sync_copy(c_hbm.at[pl.ds(start * CHUNK, base_cnt * CHUNK)],
                            idx_v)

        off = w * ncombo

        @plsc.parallel_loop(0, strip // LANES, unroll=4)
        def rebase(k):
            idx_v[pl.ds(k * LANES, LANES)] = (
                idx_v[pl.ds(k * LANES, LANES)] + off)

        def gather_start(lc, buf):
            pltpu.async_copy(
                t_rep_hbm.at[idx_v.at[pl.ds(lc * CHUNK, CHUNK)]],
                sbuf.at[pl.ds(buf * CHUNK, CHUNK), :], g_sem)

        def gather_wait(buf):
            pltpu.make_async_copy(t_rep_hbm.at[idx_v.at[pl.ds(0, CHUNK)]],
                                  sbuf.at[pl.ds(buf * CHUNK, CHUNK), :],
                                  g_sem).wait()

        def write_start(bufref, lc, buf, sem):
            pltpu.async_copy(
                bufref.at[pl.ds(buf * CHUNK, CHUNK), :],
                out_hbm.at[pl.ds((start + lc) * CHUNK, CHUNK), :],
                sem)

        def write_wait(bufref, sem):
            pltpu.make_async_copy(bufref.at[pl.ds(0, CHUNK), :],
                                  out_hbm.at[pl.ds(0, CHUNK), :],
                                  sem).wait()

        def build_chunk(lc, buf):
            @plsc.parallel_loop(0, CHUNK // LANES, unroll=1)
            def grp(g):
                cv = (idx_v[pl.ds(lc * CHUNK + g * LANES, LANES)] - off) * EMB
                rbase = buf * CHUNK + g * LANES
                for k in range(LANES):
                    c = cv[k]
                    vals = [t_v[pl.ds(c + j * LANES, LANES)]
                            for j in range(EMB // LANES)]
                    for j, v in enumerate(vals):
                        tbuf[rbase + k, pl.ds(j * LANES, LANES)] = v

        gather_start(0, 0)

        def body(j, carry):
            buf = lax.rem(j, 2)
            sc_lc = 2 * j
            tec_lc = 2 * j + 1

            @pl.when(sc_lc < count)
            def _():
                gather_wait(buf)

                @pl.when(j >= 1)
                def _():
                    write_wait(sbuf, ws_sem)

                @pl.when(sc_lc + 2 < count)
                def _():
                    gather_start(sc_lc + 2, 1 - buf)

                write_start(sbuf, sc_lc, buf, ws_sem)

            @pl.when(tec_lc < count)
            def _():
                @pl.when(j >= 2)
                def _():
                    write_wait(tbuf, wt_sem)

                build_chunk(tec_lc, buf)
                write_start(tbuf, tec_lc, buf, wt_sem)

            return carry

        lax.fori_loop(0, iters2, body, 0)
        write_wait(sbuf, ws_sem)
        write_wait(tbuf, wt_sem)
        write_wait(tbuf, wt_sem)

    return sc_build


def kernel(edge_attr, W0, W1, W2):
    num_edges = edge_attr.shape[0]
    attr = edge_attr.astype(jnp.int32)
    rows = num_edges // CHUNK
    a0 = attr[:, 0].reshape(rows, CHUNK)
    a1 = attr[:, 1].reshape(rows, CHUNK)
    a2 = attr[:, 2].reshape(rows, CHUNK)

    ncombo = W0.shape[0] * W1.shape[0] * W2.shape[0]
    t_rep, c2d = pl.pallas_call(
        _prep_body,
        out_shape=(
            jax.ShapeDtypeStruct((NUM_TILES, ncombo, EMB), jnp.float32),
            jax.ShapeDtypeStruct((rows, CHUNK), jnp.int32),
        ),
    )(W0, W1, W2, a0, a1, a2)

    return _make_sc_kernel(num_edges, ncombo)(
        t_rep.reshape(NUM_TILES * ncombo, EMB),
        t_rep[0].reshape(ncombo * EMB),
        c2d.reshape(num_edges))
